# baseline (speedup 1.0000x reference)
import functools

import jax
import jax.numpy as jnp
from jax import lax
from jax.experimental import pallas as pl
from jax.experimental.pallas import tpu as pltpu
from jax.experimental.pallas import tpu_sc as plsc

ROWS = 1024
GENES = 500
D = 100
TBL = ROWS * GENES
OUT = TBL // 2
NC, NS = 2, 16
NW = NC * NS
PER_W = OUT // NW
C = 80
CHUNKS = PER_W // C
NBUF = 4

_mesh = plsc.VectorSubcoreMesh(core_axis_name="c", subcore_axis_name="s")


def _gene_select_body(x_hbm, idx_hbm, y_hbm, idx_v, *scratch):
    bufs = scratch[:NBUF]
    gsems = scratch[NBUF:2 * NBUF]
    wsems = scratch[2 * NBUF:]

    wid = lax.axis_index("s") * NC + lax.axis_index("c")
    base0 = wid * PER_W
    pltpu.sync_copy(idx_hbm.at[wid], idx_v)

    gh = [None] * NBUF
    wh = [None] * NBUF

    def start_gather(j):
        b = j % NBUF
        gh[b] = pltpu.async_copy(x_hbm.at[idx_v.at[j]], bufs[b], gsems[b])

    def finish_step(j):
        b = j % NBUF
        gh[b].wait()
        wh[b] = pltpu.async_copy(
            bufs[b], y_hbm.at[pl.ds(base0 + j * C, C)], wsems[b]
        )

    for j in range(CHUNKS):
        b = j % NBUF
        if j >= NBUF:
            wh[b].wait()
        start_gather(j)
        jc = j - (NBUF - 1)
        if jc >= 0:
            finish_step(jc)
    for jc in range(CHUNKS - (NBUF - 1), CHUNKS):
        finish_step(jc)
    for b in range(NBUF):
        wh[b].wait()


def _make_gene_select(interpret=False):
    return functools.partial(
        pl.kernel,
        out_type=jax.ShapeDtypeStruct((OUT, D), jnp.float32),
        mesh=_mesh,
        scratch_types=(
            [pltpu.VMEM((CHUNKS, C), jnp.int32)]
            + [pltpu.VMEM((C, D), jnp.float32) for _ in range(NBUF)]
            + [pltpu.SemaphoreType.DMA for _ in range(2 * NBUF)]
        ),
        compiler_params=pltpu.CompilerParams(use_tc_tiling_on_sc=False),
        interpret=interpret,
    )(_gene_select_body)


_gene_select = _make_gene_select()


def kernel(inputs):
    table = inputs.reshape(TBL, D)
    k = jnp.arange(OUT, dtype=jnp.int32)
    idx = (k // (GENES // 2)) * GENES + (k % (GENES // 2)) * 2
    idx3 = idx.reshape(NW, CHUNKS, C)
    y = _gene_select(table, idx3)
    return y.reshape(ROWS, (GENES // 2) * D)


# speedup vs baseline: 2.8906x; 2.8906x over previous
import functools

import jax
import jax.numpy as jnp
from jax.experimental import pallas as pl
from jax.experimental.pallas import tpu as pltpu

ROWS = 1024
COLS = 50000
OUT_COLS = 25000
RB = 8


def _select_body(x_ref, o_ref):
    x = x_ref[...]
    o_ref[...] = x.reshape(RB, COLS // 200, 200)[:, :, :100].reshape(
        RB, OUT_COLS
    )


@jax.jit
def kernel(inputs):
    grid = (ROWS // RB,)
    return pl.pallas_call(
        _select_body,
        grid=grid,
        in_specs=[pl.BlockSpec((RB, COLS), lambda i: (i, 0))],
        out_specs=pl.BlockSpec((RB, OUT_COLS), lambda i: (i, 0)),
        out_shape=jax.ShapeDtypeStruct((ROWS, OUT_COLS), jnp.float32),
        compiler_params=pltpu.CompilerParams(
            dimension_semantics=("arbitrary",),
        ),
    )(inputs)
